# baseline (speedup 1.0000x reference)
import jax
import jax.numpy as jnp
from jax import lax
from jax.experimental import pallas as pl
from jax.experimental.pallas import tpu as pltpu
from jax.experimental.pallas import tpu_sc as plsc

_N = 16384
_D = 64
_L = 16
_NW = 32
_ROWS = _N // _NW
_PCOLS = 176

_mesh = plsc.VectorSubcoreMesh(core_axis_name="c", subcore_axis_name="s")


def _sc_body(mb_hbm, cen_hbm, out_hbm, x_v, cen_v, row_v):
    wid = lax.axis_index("c") * 16 + lax.axis_index("s")
    pltpu.sync_copy(cen_hbm, cen_v)
    pltpu.sync_copy(mb_hbm.at[pl.ds(wid * _ROWS, _ROWS)], x_v)

    lane = lax.iota(jnp.int32, _L)
    perms = [lane ^ k for k in (1, 2, 4, 8)]

    def allsum(v):
        for p in perms:
            v = v + v.at[p].get(mode="promise_in_bounds")
        return v

    c0 = [cen_v[0, pl.ds(16 * j, 16)] for j in range(4)]
    c1 = [cen_v[1, pl.ds(16 * j, 16)] for j in range(4)]
    d01 = [a - b for a, b in zip(c0, c1)]

    def dot4(a, b):
        return allsum(a[0] * b[0] + a[1] * b[1] + a[2] * b[2] + a[3] * b[3])

    th = dot4([a + b for a, b in zip(c0, c1)], d01) * jnp.float32(0.5)
    q0 = dot4(c0, d01)
    q1 = dot4(c1, d01)

    zv = jnp.zeros((_L,), jnp.float32)
    one = jnp.float32(1.0)

    def body(r, carry):
        cnt1, ps0, ps1, s1, sall = carry
        x = [x_v[r, pl.ds(16 * j, 16)] for j in range(4)]
        t = dot4(x, d01)
        af = jnp.where(t < th, one, jnp.float32(0.0))
        cnt1 = cnt1 + af
        ps1 = ps1 + af * jnp.abs(t - q1)
        ps0 = ps0 + (one - af) * jnp.abs(t - q0)
        s1 = tuple(s + af * xv for s, xv in zip(s1, x))
        sall = tuple(s + xv for s, xv in zip(sall, x))
        return cnt1, ps0, ps1, s1, sall

    init = (zv, zv, zv, (zv, zv, zv, zv), (zv, zv, zv, zv))
    cnt1, ps0, ps1, s1, sall = lax.fori_loop(0, _ROWS, body, init)

    for j in range(4):
        row_v[pl.ds(16 * j, 16)] = s1[j]
        row_v[pl.ds(64 + 16 * j, 16)] = sall[j]
    row_v[pl.ds(128, 16)] = cnt1
    row_v[pl.ds(144, 16)] = ps0
    row_v[pl.ds(160, 16)] = ps1
    pltpu.sync_copy(row_v, out_hbm.at[wid])


_sc_partials = pl.kernel(
    _sc_body,
    out_type=jax.ShapeDtypeStruct((_NW, _PCOLS), jnp.float32),
    mesh=_mesh,
    scratch_types=[
        pltpu.VMEM((_ROWS, _D), jnp.float32),
        pltpu.VMEM((2, _D), jnp.float32),
        pltpu.VMEM((_PCOLS,), jnp.float32),
    ],
)


def _finish_body(p_ref, cen_ref, out_ref):
    p = p_ref[...]
    cen = cen_ref[...]
    s1 = jnp.sum(p[:, 0:_D], axis=0)
    sall = jnp.sum(p[:, _D:2 * _D], axis=0)
    cnt1 = jnp.sum(p[:, 128])
    ps0 = jnp.sum(p[:, 144])
    ps1 = jnp.sum(p[:, 160])
    s0 = sall - s1
    cnt0 = jnp.float32(_N) - cnt1
    sc0 = jnp.maximum(cnt0, 1.0)
    sc1 = jnp.maximum(cnt1, 1.0)
    m0 = s0 / sc0
    m1 = s1 / sc1
    nc = 0.5 * (jnp.sqrt(jnp.sum((cen[0] - m0) ** 2))
                + jnp.sqrt(jnp.sum((cen[1] - m1) ** 2)))
    d01 = cen[0] - cen[1]
    nrm = jnp.sqrt(jnp.sum(d01 * d01)) + 1e-12
    dc = 0.5 * (ps0 / sc0 + ps1 / sc1) / nrm
    out_ref[...] = jnp.reshape(nc + dc, (1, 1))


_finish = pl.pallas_call(
    _finish_body,
    out_shape=jax.ShapeDtypeStruct((1, 1), jnp.float32),
)


def kernel(minibatch, centers):
    partials = _sc_partials(minibatch, centers)
    return _finish(partials, centers)[0, 0]


# speedup vs baseline: 1.0501x; 1.0501x over previous
import jax
import jax.numpy as jnp
from jax import lax
from jax.experimental import pallas as pl
from jax.experimental.pallas import tpu as pltpu
from jax.experimental.pallas import tpu_sc as plsc

_N = 16384
_D = 64
_L = 16
_NW = 32
_ROWS = _N // _NW
_PCOLS = 176

_mesh = plsc.VectorSubcoreMesh(core_axis_name="c", subcore_axis_name="s")


_CH = 128
_NCH = _ROWS // _CH


def _sc_body(mb_hbm, cen_hbm, out_hbm, xa, xb, cen_v, row_v, sema, semb):
    wid = lax.axis_index("c") * 16 + lax.axis_index("s")
    base = wid * _ROWS
    bufs = (xa, xb)
    sems = (sema, semb)
    copies = [
        pltpu.async_copy(mb_hbm.at[pl.ds(base + c * _CH, _CH)],
                         bufs[c % 2], sems[c % 2])
        for c in range(min(2, _NCH))
    ]
    pltpu.sync_copy(cen_hbm, cen_v)

    lane = lax.iota(jnp.int32, _L)
    perms = [lane ^ k for k in (1, 2, 4, 8)]

    def allsum(v):
        for p in perms:
            v = v + v.at[p].get(mode="promise_in_bounds")
        return v

    c0 = [cen_v[0, pl.ds(16 * j, 16)] for j in range(4)]
    c1 = [cen_v[1, pl.ds(16 * j, 16)] for j in range(4)]
    d01 = [a - b for a, b in zip(c0, c1)]

    def dot4(a, b):
        return allsum(a[0] * b[0] + a[1] * b[1] + a[2] * b[2] + a[3] * b[3])

    th = dot4([a + b for a, b in zip(c0, c1)], d01) * jnp.float32(0.5)
    q0 = dot4(c0, d01)
    q1 = dot4(c1, d01)

    zv = jnp.zeros((_L,), jnp.float32)
    one = jnp.float32(1.0)

    def make_body(x_v):
        def body(r, carry):
            cnt1, psm, ps1, s1, sall = carry
            x = [x_v[r, pl.ds(16 * j, 16)] for j in range(4)]
            t = dot4(x, d01)
            af = jnp.where(t < th, one, jnp.float32(0.0))
            cnt1 = cnt1 + af
            pm = jnp.minimum(jnp.abs(t - q0), jnp.abs(t - q1))
            psm = psm + pm
            ps1 = ps1 + af * pm
            s1 = tuple(s + af * xv for s, xv in zip(s1, x))
            sall = tuple(s + xv for s, xv in zip(sall, x))
            return cnt1, psm, ps1, s1, sall
        return body

    carry = (zv, zv, zv, (zv, zv, zv, zv), (zv, zv, zv, zv))
    for c in range(_NCH):
        copies[c].wait()
        carry = lax.fori_loop(0, _CH, make_body(bufs[c % 2]), carry)
        if c + 2 < _NCH:
            copies.append(
                pltpu.async_copy(mb_hbm.at[pl.ds(base + (c + 2) * _CH, _CH)],
                                 bufs[c % 2], sems[c % 2]))
    cnt1, psm, ps1, s1, sall = carry

    for j in range(4):
        row_v[pl.ds(16 * j, 16)] = s1[j]
        row_v[pl.ds(64 + 16 * j, 16)] = sall[j]
    row_v[pl.ds(128, 16)] = cnt1
    row_v[pl.ds(144, 16)] = psm
    row_v[pl.ds(160, 16)] = ps1
    pltpu.sync_copy(row_v, out_hbm.at[wid])


_sc_partials = pl.kernel(
    _sc_body,
    out_type=jax.ShapeDtypeStruct((_NW, _PCOLS), jnp.float32),
    mesh=_mesh,
    scratch_types=[
        pltpu.VMEM((_CH, _D), jnp.float32),
        pltpu.VMEM((_CH, _D), jnp.float32),
        pltpu.VMEM((2, _D), jnp.float32),
        pltpu.VMEM((_PCOLS,), jnp.float32),
        pltpu.SemaphoreType.DMA,
        pltpu.SemaphoreType.DMA,
    ],
)


def _finish_body(p_ref, cen_ref, out_ref):
    p = p_ref[...]
    cen = cen_ref[...]
    s1 = jnp.sum(p[:, 0:_D], axis=0)
    sall = jnp.sum(p[:, _D:2 * _D], axis=0)
    cnt1 = jnp.sum(p[:, 128])
    psm = jnp.sum(p[:, 144])
    ps1 = jnp.sum(p[:, 160])
    ps0 = psm - ps1
    s0 = sall - s1
    cnt0 = jnp.float32(_N) - cnt1
    sc0 = jnp.maximum(cnt0, 1.0)
    sc1 = jnp.maximum(cnt1, 1.0)
    m0 = s0 / sc0
    m1 = s1 / sc1
    nc = 0.5 * (jnp.sqrt(jnp.sum((cen[0] - m0) ** 2))
                + jnp.sqrt(jnp.sum((cen[1] - m1) ** 2)))
    d01 = cen[0] - cen[1]
    nrm = jnp.sqrt(jnp.sum(d01 * d01)) + 1e-12
    dc = 0.5 * (ps0 / sc0 + ps1 / sc1) / nrm
    out_ref[...] = jnp.reshape(nc + dc, (1, 1))


_finish = pl.pallas_call(
    _finish_body,
    out_shape=jax.ShapeDtypeStruct((1, 1), jnp.float32),
)


def kernel(minibatch, centers):
    partials = _sc_partials(minibatch, centers)
    return _finish(partials, centers)[0, 0]


# speedup vs baseline: 1.0690x; 1.0180x over previous
import jax
import jax.numpy as jnp
from jax import lax
from jax.experimental import pallas as pl
from jax.experimental.pallas import tpu as pltpu
from jax.experimental.pallas import tpu_sc as plsc

_N = 16384
_D = 64
_L = 16
_NW = 32
_ROWS = _N // _NW
_PCOLS = 176

_mesh = plsc.VectorSubcoreMesh(core_axis_name="c", subcore_axis_name="s")


_CH = 128
_NCH = _ROWS // _CH


def _sc_body(mb_hbm, cen_hbm, out_hbm, xa, xb, cen_v, row_v, sema, semb):
    wid = lax.axis_index("c") * 16 + lax.axis_index("s")
    base = wid * _ROWS
    bufs = (xa, xb)
    sems = (sema, semb)
    copies = [
        pltpu.async_copy(mb_hbm.at[pl.ds(base + c * _CH, _CH)],
                         bufs[c % 2], sems[c % 2])
        for c in range(min(2, _NCH))
    ]
    pltpu.sync_copy(cen_hbm, cen_v)

    lane = lax.iota(jnp.int32, _L)
    perms = [lane ^ k for k in (1, 2, 4, 8)]

    def allsum(v):
        for p in perms:
            v = v + v.at[p].get(mode="promise_in_bounds")
        return v

    c0 = [cen_v[0, pl.ds(16 * j, 16)] for j in range(4)]
    c1 = [cen_v[1, pl.ds(16 * j, 16)] for j in range(4)]
    d01 = [a - b for a, b in zip(c0, c1)]

    def dot4(a, b):
        return allsum(a[0] * b[0] + a[1] * b[1] + a[2] * b[2] + a[3] * b[3])

    th = dot4([a + b for a, b in zip(c0, c1)], d01) * jnp.float32(0.5)
    q0 = dot4(c0, d01)
    q1 = dot4(c1, d01)

    zv = jnp.zeros((_L,), jnp.float32)
    one = jnp.float32(1.0)

    m4 = lane < 4
    m8 = lane < 8
    m12 = lane < 12
    p8, p4, p2, p1 = perms[3], perms[2], perms[1], perms[0]
    row_idx = [jnp.full((_L,), 4 * k, jnp.int32) for k in range(4)]

    def gat(v, p):
        return v.at[p].get(mode="promise_in_bounds")

    def make_body(x_v):
        def body(g, carry):
            cnt1, psm, ps1, s1, sall = carry
            r = g * 4
            x = [[x_v[r + k, pl.ds(16 * j, 16)] for j in range(4)]
                 for k in range(4)]
            f = []
            for k in range(4):
                tv = (x[k][0] * d01[0] + x[k][1] * d01[1]
                      + x[k][2] * d01[2] + x[k][3] * d01[3])
                tv = tv + gat(tv, p8)
                f.append(tv + gat(tv, p4))
            u = jnp.where(m4, f[0], jnp.where(m8, f[1],
                                              jnp.where(m12, f[2], f[3])))
            u = u + gat(u, p1)
            t = u + gat(u, p2)
            af = jnp.where(t < th, one, jnp.float32(0.0))
            cnt1 = cnt1 + af
            pm = jnp.minimum(jnp.abs(t - q0), jnp.abs(t - q1))
            psm = psm + pm
            ps1 = ps1 + af * pm
            for k in range(4):
                afk = gat(af, row_idx[k])
                s1 = tuple(s + afk * xv for s, xv in zip(s1, x[k]))
                sall = tuple(s + xv for s, xv in zip(sall, x[k]))
            return cnt1, psm, ps1, s1, sall
        return body

    carry = (zv, zv, zv, (zv, zv, zv, zv), (zv, zv, zv, zv))
    for c in range(_NCH):
        copies[c].wait()
        carry = lax.fori_loop(0, _CH // 4, make_body(bufs[c % 2]), carry)
        if c + 2 < _NCH:
            copies.append(
                pltpu.async_copy(mb_hbm.at[pl.ds(base + (c + 2) * _CH, _CH)],
                                 bufs[c % 2], sems[c % 2]))
    cnt1, psm, ps1, s1, sall = carry

    for j in range(4):
        row_v[pl.ds(16 * j, 16)] = s1[j]
        row_v[pl.ds(64 + 16 * j, 16)] = sall[j]
    row_v[pl.ds(128, 16)] = cnt1
    row_v[pl.ds(144, 16)] = psm
    row_v[pl.ds(160, 16)] = ps1
    pltpu.sync_copy(row_v, out_hbm.at[wid])


_sc_partials = pl.kernel(
    _sc_body,
    out_type=jax.ShapeDtypeStruct((_NW, _PCOLS), jnp.float32),
    mesh=_mesh,
    scratch_types=[
        pltpu.VMEM((_CH, _D), jnp.float32),
        pltpu.VMEM((_CH, _D), jnp.float32),
        pltpu.VMEM((2, _D), jnp.float32),
        pltpu.VMEM((_PCOLS,), jnp.float32),
        pltpu.SemaphoreType.DMA,
        pltpu.SemaphoreType.DMA,
    ],
)


def _finish_body(p_ref, cen_ref, out_ref):
    p = p_ref[...]
    cen = cen_ref[...]
    s1 = jnp.sum(p[:, 0:_D], axis=0)
    sall = jnp.sum(p[:, _D:2 * _D], axis=0)
    cnt1 = jnp.sum(p[:, 128:144]) * 0.25
    psm = jnp.sum(p[:, 144:160]) * 0.25
    ps1 = jnp.sum(p[:, 160:176]) * 0.25
    ps0 = psm - ps1
    s0 = sall - s1
    cnt0 = jnp.float32(_N) - cnt1
    sc0 = jnp.maximum(cnt0, 1.0)
    sc1 = jnp.maximum(cnt1, 1.0)
    m0 = s0 / sc0
    m1 = s1 / sc1
    nc = 0.5 * (jnp.sqrt(jnp.sum((cen[0] - m0) ** 2))
                + jnp.sqrt(jnp.sum((cen[1] - m1) ** 2)))
    d01 = cen[0] - cen[1]
    nrm = jnp.sqrt(jnp.sum(d01 * d01)) + 1e-12
    dc = 0.5 * (ps0 / sc0 + ps1 / sc1) / nrm
    out_ref[...] = jnp.reshape(nc + dc, (1, 1))


_finish = pl.pallas_call(
    _finish_body,
    out_shape=jax.ShapeDtypeStruct((1, 1), jnp.float32),
)


def kernel(minibatch, centers):
    partials = _sc_partials(minibatch, centers)
    return _finish(partials, centers)[0, 0]
